# baseline (speedup 1.0000x reference)
import functools
import jax
import jax.numpy as jnp
from jax import lax
from jax.experimental import pallas as pl
from jax.experimental.pallas import tpu as pltpu
from jax.experimental.pallas import tpu_sc as plsc

L = 16
D = 1024
B = 4 * 4096
NC, NS = 2, 16
NW = NC * NS
BPW = B // NW
TB = 2048

_mesh = plsc.VectorSubcoreMesh(core_axis_name="c", subcore_axis_name="s")


@functools.partial(
    pl.kernel,
    out_type=(
        jax.ShapeDtypeStruct((B,), jnp.float32),
        jax.ShapeDtypeStruct((2 * D,), jnp.float32),
    ),
    mesh=_mesh,
    scratch_types=[
        pltpu.VMEM((BPW,), jnp.int32),
        pltpu.VMEM((BPW,), jnp.float32),
        pltpu.VMEM((2 * D,), jnp.float32),
        pltpu.VMEM((2 * D,), jnp.float32),
    ],
)
def _route(table_hbm, idx_hbm, w_hbm, rows_hbm, idx_v, w_v, tab_v, rp_v):
    sid = lax.axis_index("s")
    cid = lax.axis_index("c")
    wid = sid * NC + cid
    base = wid * BPW
    pltpu.sync_copy(idx_hbm.at[pl.ds(base, BPW)], idx_v)

    @plsc.parallel_loop(0, BPW, step=L, unroll=4)
    def _conv(o):
        w_v[pl.ds(o, L)] = idx_v[pl.ds(o, L)].astype(jnp.float32)

    pltpu.sync_copy(w_v, w_hbm.at[pl.ds(base, BPW)])

    @pl.when(wid == 0)
    def _rows():
        pltpu.sync_copy(table_hbm, tab_v)

        @plsc.parallel_loop(0, D, step=L, unroll=4)
        def _diff(o):
            rp_v[pl.ds(o, L)] = tab_v[pl.ds(o, L)]
            rp_v[pl.ds(D + o, L)] = (
                tab_v[pl.ds(D + o, L)] - tab_v[pl.ds(o, L)]
            )

        pltpu.sync_copy(rp_v, rows_hbm)


def _expand_body(w_ref, rows_ref, out_ref):
    r0 = rows_ref[0:1, :]
    dr = rows_ref[1:2, :]
    out_ref[...] = r0 + w_ref[...][:, None] * dr


_expand = pl.pallas_call(
    _expand_body,
    grid=(B // TB,),
    in_specs=[
        pl.BlockSpec((TB,), lambda i: (i,)),
        pl.BlockSpec((2, D), lambda i: (0, 0)),
    ],
    out_specs=pl.BlockSpec((TB, D), lambda i: (i, 0)),
    out_shape=jax.ShapeDtypeStruct((B, D), jnp.float32),
)


def kernel(segment_ids, table):
    idx = segment_ids.reshape(-1).astype(jnp.int32)
    w, rows = _route(table.reshape(-1), idx)
    out = _expand(w, rows.reshape(2, D))
    return out.reshape(segment_ids.shape + (D,))


# speedup vs baseline: 1.0267x; 1.0267x over previous
import functools
import jax
import jax.numpy as jnp
from jax import lax
from jax.experimental import pallas as pl
from jax.experimental.pallas import tpu as pltpu
from jax.experimental.pallas import tpu_sc as plsc

L = 16
D = 1024
B = 4 * 4096
NC, NS = 2, 16
NW = NC * NS
BPW = B // NW
TB = 1024

_mesh = plsc.VectorSubcoreMesh(core_axis_name="c", subcore_axis_name="s")


@functools.partial(
    pl.kernel,
    out_type=(
        jax.ShapeDtypeStruct((B,), jnp.float32),
        jax.ShapeDtypeStruct((2 * D,), jnp.float32),
    ),
    mesh=_mesh,
    scratch_types=[
        pltpu.VMEM((BPW,), jnp.int32),
        pltpu.VMEM((BPW,), jnp.float32),
        pltpu.VMEM((2 * D,), jnp.float32),
        pltpu.VMEM((2 * D,), jnp.float32),
    ],
)
def _route(table_hbm, idx_hbm, w_hbm, rows_hbm, idx_v, w_v, tab_v, rp_v):
    sid = lax.axis_index("s")
    cid = lax.axis_index("c")
    wid = sid * NC + cid
    base = wid * BPW
    pltpu.sync_copy(idx_hbm.at[pl.ds(base, BPW)], idx_v)

    @plsc.parallel_loop(0, BPW, step=L, unroll=4)
    def _conv(o):
        w_v[pl.ds(o, L)] = idx_v[pl.ds(o, L)].astype(jnp.float32)

    pltpu.sync_copy(w_v, w_hbm.at[pl.ds(base, BPW)])

    @pl.when(wid == 0)
    def _rows():
        pltpu.sync_copy(table_hbm, tab_v)

        @plsc.parallel_loop(0, D, step=L, unroll=4)
        def _diff(o):
            rp_v[pl.ds(o, L)] = tab_v[pl.ds(o, L)]
            rp_v[pl.ds(D + o, L)] = (
                tab_v[pl.ds(D + o, L)] - tab_v[pl.ds(o, L)]
            )

        pltpu.sync_copy(rp_v, rows_hbm)


def _expand_body(w_ref, rows_ref, out_ref):
    r0 = rows_ref[0:1, :]
    dr = rows_ref[1:2, :]
    out_ref[...] = r0 + w_ref[...][:, None] * dr


_expand = pl.pallas_call(
    _expand_body,
    grid=(B // TB,),
    in_specs=[
        pl.BlockSpec((TB,), lambda i: (i,)),
        pl.BlockSpec((2, D), lambda i: (0, 0)),
    ],
    out_specs=pl.BlockSpec((TB, D), lambda i: (i, 0)),
    out_shape=jax.ShapeDtypeStruct((B, D), jnp.float32),
)


def kernel(segment_ids, table):
    idx = segment_ids.reshape(-1).astype(jnp.int32)
    w, rows = _route(table.reshape(-1), idx)
    out = _expand(w, rows.reshape(2, D))
    return out.reshape(segment_ids.shape + (D,))


# speedup vs baseline: 1.0817x; 1.0536x over previous
import functools
import jax
import jax.numpy as jnp
from jax import lax
from jax.experimental import pallas as pl
from jax.experimental.pallas import tpu as pltpu
from jax.experimental.pallas import tpu_sc as plsc

L = 16
D = 1024
B = 4 * 4096
NC, NS = 2, 16
NW = NC * NS
BPW = B // NW
TB = 1024

_mesh = plsc.VectorSubcoreMesh(core_axis_name="c", subcore_axis_name="s")


@functools.partial(
    pl.kernel,
    out_type=jax.ShapeDtypeStruct((B,), jnp.float32),
    mesh=_mesh,
    scratch_types=[
        pltpu.VMEM((BPW,), jnp.int32),
        pltpu.VMEM((BPW,), jnp.float32),
    ],
)
def _route(idx_hbm, w_hbm, idx_v, w_v):
    wid = lax.axis_index("s") * NC + lax.axis_index("c")
    base = wid * BPW
    pltpu.sync_copy(idx_hbm.at[pl.ds(base, BPW)], idx_v)

    @plsc.parallel_loop(0, BPW, step=L, unroll=4)
    def _conv(o):
        w_v[pl.ds(o, L)] = idx_v[pl.ds(o, L)].astype(jnp.float32)

    pltpu.sync_copy(w_v, w_hbm.at[pl.ds(base, BPW)])


def _expand_body(w_ref, table_ref, out_ref):
    r0 = table_ref[0:1, :]
    r1 = table_ref[1:2, :]
    out_ref[...] = jnp.where(w_ref[...][:, None] != 0.0, r1, r0)


_expand = pl.pallas_call(
    _expand_body,
    grid=(B // TB,),
    in_specs=[
        pl.BlockSpec((TB,), lambda i: (i,)),
        pl.BlockSpec((2, D), lambda i: (0, 0)),
    ],
    out_specs=pl.BlockSpec((TB, D), lambda i: (i, 0)),
    out_shape=jax.ShapeDtypeStruct((B, D), jnp.float32),
)


def kernel(segment_ids, table):
    idx = segment_ids.reshape(-1).astype(jnp.int32)
    w = _route(idx)
    out = _expand(w, table)
    return out.reshape(segment_ids.shape + (D,))
